# Initial kernel scaffold; baseline (speedup 1.0000x reference)
import jax
import jax.numpy as jnp
from jax.experimental import pallas as pl


def kernel(x_query, vis_mark, g_p_0, g_p_1, e_p_2, e_k_2, e_p_3, e_k_3, e_p_4, e_k_4):
    raise NotImplementedError("write your pallas kernel here")


# speedup vs baseline: 1.8021x; 1.8021x over previous
import functools

import jax
import jax.numpy as jnp
from jax import lax
from jax.experimental import pallas as pl
from jax.experimental.pallas import tpu as pltpu
from jax.experimental.pallas import tpu_sc as plsc

_B = 128
_POOL = 1000
_K = 5
_EPL = 8
_D = 768
_NC = 2
_NS = 16
_NW = _NC * _NS
_NL = 3
_SUBROWS = _B * _K * _EPL
_RPW = _SUBROWS // _NW
_CHUNK = 40
_NCHUNK = _RPW // _CHUNK
_TOT_CHUNKS = _NL * _NCHUNK


def _score_topk_body(x2, x3, x4, k2, k3, k4, o_ref):
    lane = lax.broadcasted_iota(jnp.int32, (_B, _K * _EPL), 1)
    colid = lax.broadcasted_iota(jnp.int32, (_B, _POOL), 1)
    for i, (x_ref, k_ref) in enumerate(((x2, k2), (x3, k3), (x4, k4))):
        kmat = k_ref[...]
        kn = jnp.maximum(jnp.sqrt(jnp.sum(kmat * kmat, axis=1, keepdims=True)),
                         1e-12)
        nk = kmat / kn
        x = x_ref[...]
        qn = jnp.maximum(jnp.sqrt(jnp.sum(x * x, axis=1, keepdims=True)),
                         1e-12)
        q = x / qn
        s = lax.dot_general(q, nk, (((1,), (1,)), ((), ())),
                            preferred_element_type=jnp.float32)
        acc = jnp.zeros((_B, _K * _EPL), jnp.int32)
        for t in range(_K):
            m = jnp.max(s, axis=1, keepdims=True)
            idx = jnp.min(jnp.where(s == m, colid, jnp.int32(2**30)),
                          axis=1, keepdims=True)
            acc = jnp.where(lane // _EPL == t, idx * _EPL + lane % _EPL, acc)
            s = jnp.where(colid == idx, -jnp.inf, s)
        o_ref[i] = acc


def _score_topk(x2, x3, x4, k2, k3, k4, interpret=False):
    return pl.pallas_call(
        _score_topk_body,
        out_shape=jax.ShapeDtypeStruct((_NL, _B, _K * _EPL), jnp.int32),
        interpret=interpret,
    )(x2, x3, x4, k2, k3, k4)


def _make_gather():
    mesh = plsc.VectorSubcoreMesh(core_axis_name="c", subcore_axis_name="s",
                                  num_cores=_NC, num_subcores=_NS)

    @functools.partial(
        pl.kernel,
        mesh=mesh,
        out_type=[jax.ShapeDtypeStruct((_SUBROWS, _D), jnp.float32)] * _NL,
        scratch_types=[
            pltpu.VMEM((_TOT_CHUNKS, _CHUNK), jnp.int32),
            pltpu.VMEM((_CHUNK, _D), jnp.float32),
            pltpu.VMEM((_CHUNK, _D), jnp.float32),
            pltpu.SemaphoreType.DMA,
            pltpu.SemaphoreType.DMA,
            pltpu.SemaphoreType.DMA,
            pltpu.SemaphoreType.DMA,
        ],
    )
    def gather(t2, t3, t4, idx_hbm, o2, o3, o4,
               idx_v, bufa, bufb, gsa, gsb, ssa, ssb):
        wid = lax.axis_index("s") * _NC + lax.axis_index("c")
        pltpu.sync_copy(idx_hbm.at[wid], idx_v)
        tabs = (t2, t3, t4)
        outs = (o2, o3, o4)
        bufs = (bufa, bufb)
        gsems = (gsa, gsb)
        ssems = (ssa, ssb)
        base = wid * _RPW

        def start_gather(c):
            l = c // _NCHUNK
            cp = pltpu.make_async_copy(
                tabs[l].at[idx_v.at[c]], bufs[c % 2], gsems[c % 2])
            cp.start()
            return cp

        def start_scatter(c):
            l, cc = divmod(c, _NCHUNK)
            cp = pltpu.make_async_copy(
                bufs[c % 2],
                outs[l].at[pl.ds(base + cc * _CHUNK, _CHUNK)],
                ssems[c % 2])
            cp.start()
            return cp

        gs = [None] * _TOT_CHUNKS
        ss = [None] * _TOT_CHUNKS
        for c in range(_TOT_CHUNKS):
            if c >= 2:
                ss[c - 2].wait()
            gs[c] = start_gather(c)
            if c >= 1:
                gs[c - 1].wait()
                ss[c - 1] = start_scatter(c - 1)
        gs[_TOT_CHUNKS - 1].wait()
        ss[_TOT_CHUNKS - 1] = start_scatter(_TOT_CHUNKS - 1)
        ss[_TOT_CHUNKS - 2].wait()
        ss[_TOT_CHUNKS - 1].wait()

    return gather


@functools.lru_cache(maxsize=1)
def _gather_cached():
    return _make_gather()


def kernel(x_query, vis_mark, g_p_0, g_p_1, e_p_2, e_k_2, e_p_3, e_k_3,
           e_p_4, e_k_4):
    x2 = x_query[:, 2, :]
    x3 = x_query[:, 3, :]
    x4 = x_query[:, 4, :]
    sub = _score_topk(x2, x3, x4, e_k_2, e_k_3, e_k_4)
    idx = (sub.reshape(_NL, _NW, _RPW)
              .transpose(1, 0, 2)
              .reshape(_NW, _TOT_CHUNKS, _CHUNK))
    t2 = e_p_2.reshape(_POOL * _EPL, _D)
    t3 = e_p_3.reshape(_POOL * _EPL, _D)
    t4 = e_p_4.reshape(_POOL * _EPL, _D)
    o2, o3, o4 = _gather_cached()(t2, t3, t4, idx)
    out2 = o2.reshape(_B, _K, _EPL, _D)
    out3 = o3.reshape(_B, _K, _EPL, _D)
    out4 = o4.reshape(_B, _K, _EPL, _D)
    out0 = jnp.broadcast_to(g_p_0[None], (_B,) + g_p_0.shape)
    out1 = jnp.broadcast_to(g_p_1[None], (_B,) + g_p_1.shape)
    loss = jnp.zeros((), jnp.float32)
    return (out0, out1, out2, out3, out4, loss)
